# baseline (speedup 1.0000x reference)
import functools

import jax
import jax.numpy as jnp
from jax import lax
from jax.experimental import pallas as pl
from jax.experimental.pallas import tpu as pltpu
from jax.experimental.pallas import tpu_sc as plsc

_N, _E, _D, _R = 10000, 320000, 128, 8
_NC, _NS = 2, 16
_NW = _NC * _NS
_EPW = _E // _NW
_CH = 80
_NCHUNK = _EPW // _CH
_SEGS = _N * _R
_SEG_P = 80128

_mesh = plsc.VectorSubcoreMesh(core_axis_name="c", subcore_axis_name="s")


def _xr_body(x_ref, w_ref, o_ref):
    o_ref[0] = jnp.dot(x_ref[...], w_ref[0],
                       preferred_element_type=jnp.float32)


def _compute_xr(x, W_rel):
    tn = 2000
    return pl.pallas_call(
        _xr_body,
        grid=(_R, _N // tn),
        in_specs=[
            pl.BlockSpec((tn, _D), lambda r, n: (n, 0)),
            pl.BlockSpec((1, _D, _D), lambda r, n: (r, 0, 0)),
        ],
        out_specs=pl.BlockSpec((1, tn, _D), lambda r, n: (r, n, 0)),
        out_shape=jax.ShapeDtypeStruct((_R, _N, _D), jnp.float32),
    )(x, W_rel)


def _count_body(dst_hbm, et_hbm, zc_hbm, cnt_out,
                dst_v, et_v, seg2d, ones_v, stage_v, cnt_sp):
    c = lax.axis_index("c")
    s = lax.axis_index("s")
    wid = c * _NS + s
    zsl = _SEG_P // _NS
    pltpu.sync_copy(zc_hbm.at[pl.ds(0, zsl)], stage_v)
    pltpu.sync_copy(stage_v, cnt_sp.at[pl.ds(s * zsl, zsl)])

    base = wid * _EPW
    pltpu.sync_copy(dst_hbm.at[pl.ds(base, _EPW)], dst_v)
    pltpu.sync_copy(et_hbm.at[pl.ds(base, _EPW)], et_v)

    def segstep(k, carry):
        sl = pl.ds(k * 16, 16)
        seg2d[0, sl] = dst_v[sl] * _R + et_v[sl]
        ones_v[sl] = jnp.ones((16,), jnp.float32)
        return carry

    lax.fori_loop(0, _EPW // 16, segstep, 0)
    plsc.subcore_barrier()
    pltpu.sync_copy(ones_v, cnt_sp.at[seg2d.at[0]], add=True)
    plsc.subcore_barrier()
    pltpu.sync_copy(cnt_sp.at[pl.ds(s * zsl, zsl)], stage_v)
    pltpu.sync_copy(stage_v, cnt_out.at[pl.ds(c * _SEG_P + s * zsl, zsl)])


_count = functools.partial(
    pl.kernel,
    out_type=jax.ShapeDtypeStruct((_NC * _SEG_P,), jnp.float32),
    mesh=_mesh,
    scratch_types=[
        pltpu.VMEM((_EPW,), jnp.int32),
        pltpu.VMEM((_EPW,), jnp.int32),
        pltpu.VMEM((1, _EPW), jnp.int32),
        pltpu.VMEM((_EPW,), jnp.float32),
        pltpu.VMEM((_SEG_P // _NS,), jnp.float32),
        pltpu.VMEM_SHARED((_SEG_P,), jnp.float32),
    ],
)(_count_body)


def _inv_body(c_ref, o_ref):
    c = c_ref[0] + c_ref[1]
    o_ref[...] = 1.0 / jnp.maximum(c, 1.0)


def _compute_inv(cnt2):
    return pl.pallas_call(
        _inv_body,
        out_shape=jax.ShapeDtypeStruct((_SEG_P // 128, 128), jnp.float32),
    )(cnt2.reshape(_NC, _SEG_P // 128, 128))


def _agg_body(xr_hbm, src_hbm, dst_hbm, et_hbm, inv_hbm, zn_hbm, out_hbm,
              rows0, rows1, srcm, etm, dstm, gidx3d, seg3d, dst3d, w0, w1,
              stage_v, stage1d, semg0, semg1, semw0, semw1, semm,
              sems0, sems1, agg_sp, inv_sp):
    c = lax.axis_index("c")
    s = lax.axis_index("s")
    wid = c * _NS + s
    pltpu.sync_copy(zn_hbm, stage_v)
    for i in range(10):
        row0 = s * 640 + i * 64

        @pl.when(row0 + 64 <= _N)
        def _():
            pltpu.sync_copy(stage_v, agg_sp.at[pl.ds(row0, 64)])

    @pl.when(s == _NS - 1)
    def _():
        pltpu.sync_copy(stage_v.at[pl.ds(0, 16)],
                        agg_sp.at[pl.ds(_N - 16, 16)])

    zsl = _SEG_P // _NS
    pltpu.sync_copy(inv_hbm.at[pl.ds(s * zsl, zsl)], stage1d)
    pltpu.sync_copy(stage1d, inv_sp.at[pl.ds(s * zsl, zsl)])
    plsc.subcore_barrier()

    base = wid * _EPW
    rows = (rows0, rows1)
    semg = (semg0, semg1)
    semw = (semw0, semw1)
    w = (w0, w1)

    def fire_meta(j, p):
        off = base + j * 2 * _CH
        ring = pl.ds(p * 2 * _CH, 2 * _CH)
        pltpu.async_copy(src_hbm.at[pl.ds(off, 2 * _CH)], srcm.at[ring], semm)
        pltpu.async_copy(et_hbm.at[pl.ds(off, 2 * _CH)], etm.at[ring], semm)
        pltpu.async_copy(dst_hbm.at[pl.ds(off, 2 * _CH)], dstm.at[ring], semm)

    def wait_meta_index(j, p):
        off = base + j * 2 * _CH
        ring = pl.ds(p * 2 * _CH, 2 * _CH)
        pltpu.make_async_copy(src_hbm.at[pl.ds(off, 2 * _CH)], srcm.at[ring],
                              semm).wait()
        pltpu.make_async_copy(et_hbm.at[pl.ds(off, 2 * _CH)], etm.at[ring],
                              semm).wait()
        pltpu.make_async_copy(dst_hbm.at[pl.ds(off, 2 * _CH)], dstm.at[ring],
                              semm).wait()
        for q in range(2):
            for k in range(_CH // 16):
                sl = pl.ds(p * 2 * _CH + q * _CH + k * 16, 16)
                dsl = pl.ds(k * 16, 16)
                e = etm[sl]
                d = dstm[sl]
                gidx3d[p, q, dsl] = e * _N + srcm[sl]
                seg3d[p, q, dsl] = d * _R + e
                dst3d[p, q, dsl] = d

    def fire_rows(b, p):
        pltpu.async_copy(xr_hbm.at[gidx3d.at[p, b]], rows[b], semg[b])
        pltpu.async_copy(inv_sp.at[seg3d.at[p, b]], w[b], semw[b])

    sems = (sems0, sems1)

    def consume(b, p):
        pltpu.make_async_copy(xr_hbm.at[gidx3d.at[p, b]], rows[b],
                              semg[b]).wait()
        pltpu.make_async_copy(inv_sp.at[seg3d.at[p, b]], w[b],
                              semw[b]).wait()
        for k in range(_CH // 16):
            w16 = w[b][pl.ds(k * 16, 16)]
            for jj in range(16):
                j = k * 16 + jj
                wv = w16[jj]
                for v in range(_D // 16):
                    vsl = pl.ds(v * 16, 16)
                    rows[b][j, vsl] = rows[b][j, vsl] * wv
        pltpu.async_copy(rows[b], agg_sp.at[dst3d.at[p, b]], sems[b],
                         add=True)

    def wait_scatter(b, p):
        pltpu.make_async_copy(rows[b], agg_sp.at[dst3d.at[p, b]],
                              sems[b]).wait()

    fire_meta(0, 0)
    wait_meta_index(0, 0)
    fire_rows(0, 0)
    fire_rows(1, 0)
    fire_meta(1, 1)

    def subbody(j, p):
        g = 2 * j
        consume(0, p)

        @pl.when(g + 2 < _NCHUNK)
        def _():
            wait_meta_index(j + 1, 1 - p)

        @pl.when(g + 1 < _NCHUNK)
        def _():
            consume(1, p)

        @pl.when(g + 2 < _NCHUNK)
        def _():
            wait_scatter(0, p)
            fire_rows(0, 1 - p)

        @pl.when(2 * (j + 2) < _NCHUNK)
        def _():
            fire_meta(j + 2, p)

        @pl.when(g + 3 < _NCHUNK)
        def _():
            wait_scatter(1, p)
            fire_rows(1, 1 - p)

    def dpair(t, carry):
        subbody(2 * t, 0)
        subbody(2 * t + 1, 1)
        return carry

    lax.fori_loop(0, 31, dpair, 0)
    consume(0, 0)
    wait_scatter(1, 1)
    wait_scatter(0, 0)
    plsc.subcore_barrier()
    for i in range(10):
        row0 = s * 640 + i * 64

        @pl.when(row0 + 64 <= _N)
        def _():
            pltpu.sync_copy(agg_sp.at[pl.ds(row0, 64)], stage_v)
            pltpu.sync_copy(stage_v, out_hbm.at[pl.ds(c * _N + row0, 64)])

    @pl.when(s == _NS - 1)
    def _():
        pltpu.sync_copy(agg_sp.at[pl.ds(_N - 16, 16)], stage_v.at[pl.ds(0, 16)])
        pltpu.sync_copy(stage_v.at[pl.ds(0, 16)],
                        out_hbm.at[pl.ds(c * _N + _N - 16, 16)])


_aggregate = functools.partial(
    pl.kernel,
    out_type=jax.ShapeDtypeStruct((_NC * _N, _D), jnp.float32),
    mesh=_mesh,
    scratch_types=[
        pltpu.VMEM((_CH, _D), jnp.float32),
        pltpu.VMEM((_CH, _D), jnp.float32),
        pltpu.VMEM((4 * _CH,), jnp.int32),
        pltpu.VMEM((4 * _CH,), jnp.int32),
        pltpu.VMEM((4 * _CH,), jnp.int32),
        pltpu.VMEM((2, 2, _CH), jnp.int32),
        pltpu.VMEM((2, 2, _CH), jnp.int32),
        pltpu.VMEM((2, 2, _CH), jnp.int32),
        pltpu.VMEM((_CH,), jnp.float32),
        pltpu.VMEM((_CH,), jnp.float32),
        pltpu.VMEM((64, _D), jnp.float32),
        pltpu.VMEM((_SEG_P // _NS,), jnp.float32),
        pltpu.SemaphoreType.DMA,
        pltpu.SemaphoreType.DMA,
        pltpu.SemaphoreType.DMA,
        pltpu.SemaphoreType.DMA,
        pltpu.SemaphoreType.DMA,
        pltpu.SemaphoreType.DMA,
        pltpu.SemaphoreType.DMA,
        pltpu.VMEM_SHARED((_N, _D), jnp.float32),
        pltpu.VMEM_SHARED((_SEG_P,), jnp.float32),
    ],
    compiler_params=pltpu.CompilerParams(needs_layout_passes=False),
)(_agg_body)


def _final_body(p0, p1, x_ref, wr_ref, wg_ref, b_ref, bg_ref, o_ref):
    x = x_ref[...]
    nodes_ = (p0[...] + p1[...]
              + jnp.dot(x, wr_ref[...], preferred_element_type=jnp.float32)
              + b_ref[0])
    z = (jnp.dot(nodes_, wg_ref[0:_D, :], preferred_element_type=jnp.float32)
         + jnp.dot(x, wg_ref[_D:2 * _D, :], preferred_element_type=jnp.float32)
         + bg_ref[0])
    g = jax.nn.sigmoid(z)
    o_ref[...] = g * jnp.tanh(nodes_) + (1.0 - g) * x


def _final(p0, p1, x, W_root, Wg, bias, bg):
    tn = 2000
    full = lambda n: (0, 0)
    return pl.pallas_call(
        _final_body,
        grid=(_N // tn,),
        in_specs=[
            pl.BlockSpec((tn, _D), lambda n: (n, 0)),
            pl.BlockSpec((tn, _D), lambda n: (n, 0)),
            pl.BlockSpec((tn, _D), lambda n: (n, 0)),
            pl.BlockSpec((_D, _D), full),
            pl.BlockSpec((2 * _D, _D), full),
            pl.BlockSpec((1, _D), full),
            pl.BlockSpec((1, _D), full),
        ],
        out_specs=pl.BlockSpec((tn, _D), lambda n: (n, 0)),
        out_shape=jax.ShapeDtypeStruct((_N, _D), jnp.float32),
    )(p0, p1, x, W_root, Wg, bias, bg)


def kernel(nodes, edges, edge_types, W_rel, W_root, bias, Wg, bg):
    x = nodes
    src = edges[0]
    dst = edges[1]
    et = edge_types

    xr = _compute_xr(x, W_rel).reshape(_R * _N, _D)
    zc = jnp.zeros((_SEG_P // _NS,), jnp.float32)
    cnt2 = _count(dst, et, zc).reshape(_NC, _SEG_P)
    inv = _compute_inv(cnt2).reshape(_SEG_P)
    zn = jnp.zeros((64, _D), jnp.float32)
    parts = _aggregate(xr, src, dst, et, inv, zn).reshape(_NC, _N, _D)
    return _final(parts[0], parts[1], x, W_root, Wg,
                  bias.reshape(1, _D), bg.reshape(1, _D))


# speedup vs baseline: 1.1553x; 1.1553x over previous
import functools

import jax
import jax.numpy as jnp
from jax import lax
from jax.experimental import pallas as pl
from jax.experimental.pallas import tpu as pltpu
from jax.experimental.pallas import tpu_sc as plsc

_N, _E, _D, _R = 10000, 320000, 128, 8
_NC, _NS = 2, 16
_NW = _NC * _NS
_EPW = _E // _NW
_CH = 80
_NCHUNK = _EPW // _CH
_SEGS = _N * _R
_SEG_P = 80128

_mesh = plsc.VectorSubcoreMesh(core_axis_name="c", subcore_axis_name="s")


def _xr_body(x_ref, w_ref, o_ref):
    o_ref[0] = jnp.dot(x_ref[...], w_ref[0],
                       preferred_element_type=jnp.float32)


def _compute_xr(x, W_rel):
    tn = 2000
    return pl.pallas_call(
        _xr_body,
        grid=(_R, _N // tn),
        in_specs=[
            pl.BlockSpec((tn, _D), lambda r, n: (n, 0)),
            pl.BlockSpec((1, _D, _D), lambda r, n: (r, 0, 0)),
        ],
        out_specs=pl.BlockSpec((1, tn, _D), lambda r, n: (r, n, 0)),
        out_shape=jax.ShapeDtypeStruct((_R, _N, _D), jnp.float32),
    )(x, W_rel)


def _count_body(dst_hbm, et_hbm, zc_hbm, cnt_out,
                dst_v, et_v, seg2d, ones_v, stage_v, cnt_sp):
    c = lax.axis_index("c")
    s = lax.axis_index("s")
    wid = c * _NS + s
    zsl = _SEG_P // _NS
    pltpu.sync_copy(zc_hbm.at[pl.ds(0, zsl)], stage_v)
    pltpu.sync_copy(stage_v, cnt_sp.at[pl.ds(s * zsl, zsl)])

    base = wid * _EPW
    pltpu.sync_copy(dst_hbm.at[pl.ds(base, _EPW)], dst_v)
    pltpu.sync_copy(et_hbm.at[pl.ds(base, _EPW)], et_v)

    def segstep(k, carry):
        sl = pl.ds(k * 16, 16)
        seg2d[0, sl] = dst_v[sl] * _R + et_v[sl]
        ones_v[sl] = jnp.ones((16,), jnp.float32)
        return carry

    lax.fori_loop(0, _EPW // 16, segstep, 0)
    plsc.subcore_barrier()
    pltpu.sync_copy(ones_v, cnt_sp.at[seg2d.at[0]], add=True)
    plsc.subcore_barrier()
    pltpu.sync_copy(cnt_sp.at[pl.ds(s * zsl, zsl)], stage_v)
    pltpu.sync_copy(stage_v, cnt_out.at[pl.ds(c * _SEG_P + s * zsl, zsl)])


_count = functools.partial(
    pl.kernel,
    out_type=jax.ShapeDtypeStruct((_NC * _SEG_P,), jnp.float32),
    mesh=_mesh,
    scratch_types=[
        pltpu.VMEM((_EPW,), jnp.int32),
        pltpu.VMEM((_EPW,), jnp.int32),
        pltpu.VMEM((1, _EPW), jnp.int32),
        pltpu.VMEM((_EPW,), jnp.float32),
        pltpu.VMEM((_SEG_P // _NS,), jnp.float32),
        pltpu.VMEM_SHARED((_SEG_P,), jnp.float32),
    ],
)(_count_body)


def _inv_body(c_ref, o_ref):
    c = c_ref[0] + c_ref[1]
    o_ref[...] = 1.0 / jnp.maximum(c, 1.0)


def _compute_inv(cnt2):
    return pl.pallas_call(
        _inv_body,
        out_shape=jax.ShapeDtypeStruct((_SEG_P // 128, 128), jnp.float32),
    )(cnt2.reshape(_NC, _SEG_P // 128, 128))


def _agg_body(xr_hbm, src_hbm, dst_hbm, et_hbm, inv_hbm, zn_hbm, out_hbm,
              rows0, rows1, srcm, etm, dstm, gidx3d, seg3d, dstA, dstB,
              w0, w1, stage_v, stage1d, semg0, semg1, semw0, semw1, semm,
              sems0, sems1, agg_sp, inv_sp):
    c = lax.axis_index("c")
    s = lax.axis_index("s")
    wid = c * _NS + s
    pltpu.sync_copy(zn_hbm, stage_v)
    for i in range(10):
        row0 = s * 640 + i * 64

        @pl.when(row0 + 64 <= _N)
        def _():
            pltpu.sync_copy(stage_v, agg_sp.at[pl.ds(row0, 64)])

    @pl.when(s == _NS - 1)
    def _():
        pltpu.sync_copy(stage_v.at[pl.ds(0, 16)],
                        agg_sp.at[pl.ds(_N - 16, 16)])

    zsl = _SEG_P // _NS
    pltpu.sync_copy(inv_hbm.at[pl.ds(s * zsl, zsl)], stage1d)
    pltpu.sync_copy(stage1d, inv_sp.at[pl.ds(s * zsl, zsl)])
    plsc.subcore_barrier()

    base = wid * _EPW
    rows = (rows0, rows1)
    semg = (semg0, semg1)
    semw = (semw0, semw1)
    w = (w0, w1)

    def fire_meta(j, p):
        off = base + j * 2 * _CH
        ring = pl.ds(p * 2 * _CH, 2 * _CH)
        pltpu.async_copy(src_hbm.at[pl.ds(off, 2 * _CH)], srcm.at[ring], semm)
        pltpu.async_copy(et_hbm.at[pl.ds(off, 2 * _CH)], etm.at[ring], semm)
        pltpu.async_copy(dst_hbm.at[pl.ds(off, 2 * _CH)], dstm.at[ring], semm)

    def wait_meta_index(j, p):
        off = base + j * 2 * _CH
        ring = pl.ds(p * 2 * _CH, 2 * _CH)
        pltpu.make_async_copy(src_hbm.at[pl.ds(off, 2 * _CH)], srcm.at[ring],
                              semm).wait()
        pltpu.make_async_copy(et_hbm.at[pl.ds(off, 2 * _CH)], etm.at[ring],
                              semm).wait()
        pltpu.make_async_copy(dst_hbm.at[pl.ds(off, 2 * _CH)], dstm.at[ring],
                              semm).wait()
        for q in range(2):
            for k in range(_CH // 16):
                sl = pl.ds(p * 2 * _CH + q * _CH + k * 16, 16)
                dsl = pl.ds(k * 16, 16)
                e = etm[sl]
                d = dstm[sl]
                gidx3d[p, q, dsl] = e * _N + srcm[sl]
                seg3d[p, q, dsl] = d * _R + e
                if k < 3:
                    dstA[p, q, pl.ds(k * 16, 16)] = d
                else:
                    dstB[p, q, pl.ds((k - 3) * 16, 16)] = d

    def fire_rows(b, p):
        pltpu.async_copy(xr_hbm.at[gidx3d.at[p, b]], rows[b], semg[b])
        pltpu.async_copy(inv_sp.at[seg3d.at[p, b]], w[b], semw[b])

    sems = (sems0, sems1)

    def _mult(b, k):
        w16 = w[b][pl.ds(k * 16, 16)]
        for jj in range(16):
            j = k * 16 + jj
            wv = w16[jj]
            for v in range(_D // 16):
                vsl = pl.ds(v * 16, 16)
                rows[b][j, vsl] = rows[b][j, vsl] * wv

    def consume(b, p):
        pltpu.make_async_copy(xr_hbm.at[gidx3d.at[p, b]], rows[b],
                              semg[b]).wait()
        pltpu.make_async_copy(inv_sp.at[seg3d.at[p, b]], w[b],
                              semw[b]).wait()
        for k in range(3):
            _mult(b, k)
        pltpu.async_copy(rows[b].at[pl.ds(0, 48)], agg_sp.at[dstA.at[p, b]],
                         sems[b], add=True)
        for k in range(3, _CH // 16):
            _mult(b, k)
        pltpu.async_copy(rows[b].at[pl.ds(48, 32)], agg_sp.at[dstB.at[p, b]],
                         sems[b], add=True)

    def wait_scatter(b, p):
        pltpu.make_async_copy(rows[b].at[pl.ds(0, 48)],
                              agg_sp.at[dstA.at[p, b]], sems[b]).wait()
        pltpu.make_async_copy(rows[b].at[pl.ds(48, 32)],
                              agg_sp.at[dstB.at[p, b]], sems[b]).wait()

    fire_meta(0, 0)
    wait_meta_index(0, 0)
    fire_rows(0, 0)
    fire_rows(1, 0)
    fire_meta(1, 1)

    def subbody(j, p):
        g = 2 * j
        consume(0, p)

        @pl.when(g + 2 < _NCHUNK)
        def _():
            wait_meta_index(j + 1, 1 - p)
            wait_scatter(0, p)
            fire_rows(0, 1 - p)

        @pl.when(g + 1 < _NCHUNK)
        def _():
            consume(1, p)

            @pl.when(g + 3 < _NCHUNK)
            def _():
                wait_scatter(1, p)
                fire_rows(1, 1 - p)

        @pl.when(2 * (j + 2) < _NCHUNK)
        def _():
            fire_meta(j + 2, p)

    def dpair(t, carry):
        subbody(2 * t, 0)
        subbody(2 * t + 1, 1)
        return carry

    lax.fori_loop(0, 31, dpair, 0)
    consume(0, 0)
    wait_scatter(1, 1)
    wait_scatter(0, 0)
    plsc.subcore_barrier()
    for i in range(10):
        row0 = s * 640 + i * 64

        @pl.when(row0 + 64 <= _N)
        def _():
            pltpu.sync_copy(agg_sp.at[pl.ds(row0, 64)], stage_v)
            pltpu.sync_copy(stage_v, out_hbm.at[pl.ds(c * _N + row0, 64)])

    @pl.when(s == _NS - 1)
    def _():
        pltpu.sync_copy(agg_sp.at[pl.ds(_N - 16, 16)], stage_v.at[pl.ds(0, 16)])
        pltpu.sync_copy(stage_v.at[pl.ds(0, 16)],
                        out_hbm.at[pl.ds(c * _N + _N - 16, 16)])


_aggregate = functools.partial(
    pl.kernel,
    out_type=jax.ShapeDtypeStruct((_NC * _N, _D), jnp.float32),
    mesh=_mesh,
    scratch_types=[
        pltpu.VMEM((_CH, _D), jnp.float32),
        pltpu.VMEM((_CH, _D), jnp.float32),
        pltpu.VMEM((4 * _CH,), jnp.int32),
        pltpu.VMEM((4 * _CH,), jnp.int32),
        pltpu.VMEM((4 * _CH,), jnp.int32),
        pltpu.VMEM((2, 2, _CH), jnp.int32),
        pltpu.VMEM((2, 2, _CH), jnp.int32),
        pltpu.VMEM((2, 2, 48), jnp.int32),
        pltpu.VMEM((2, 2, 32), jnp.int32),
        pltpu.VMEM((_CH,), jnp.float32),
        pltpu.VMEM((_CH,), jnp.float32),
        pltpu.VMEM((64, _D), jnp.float32),
        pltpu.VMEM((_SEG_P // _NS,), jnp.float32),
        pltpu.SemaphoreType.DMA,
        pltpu.SemaphoreType.DMA,
        pltpu.SemaphoreType.DMA,
        pltpu.SemaphoreType.DMA,
        pltpu.SemaphoreType.DMA,
        pltpu.SemaphoreType.DMA,
        pltpu.SemaphoreType.DMA,
        pltpu.VMEM_SHARED((_N, _D), jnp.float32),
        pltpu.VMEM_SHARED((_SEG_P,), jnp.float32),
    ],
    compiler_params=pltpu.CompilerParams(needs_layout_passes=False),
)(_agg_body)


def _final_body(p0, p1, x_ref, wr_ref, wg_ref, b_ref, bg_ref, o_ref):
    x = x_ref[...]
    nodes_ = (p0[...] + p1[...]
              + jnp.dot(x, wr_ref[...], preferred_element_type=jnp.float32)
              + b_ref[0])
    z = (jnp.dot(nodes_, wg_ref[0:_D, :], preferred_element_type=jnp.float32)
         + jnp.dot(x, wg_ref[_D:2 * _D, :], preferred_element_type=jnp.float32)
         + bg_ref[0])
    g = jax.nn.sigmoid(z)
    o_ref[...] = g * jnp.tanh(nodes_) + (1.0 - g) * x


def _final(p0, p1, x, W_root, Wg, bias, bg):
    tn = 2000
    full = lambda n: (0, 0)
    return pl.pallas_call(
        _final_body,
        grid=(_N // tn,),
        in_specs=[
            pl.BlockSpec((tn, _D), lambda n: (n, 0)),
            pl.BlockSpec((tn, _D), lambda n: (n, 0)),
            pl.BlockSpec((tn, _D), lambda n: (n, 0)),
            pl.BlockSpec((_D, _D), full),
            pl.BlockSpec((2 * _D, _D), full),
            pl.BlockSpec((1, _D), full),
            pl.BlockSpec((1, _D), full),
        ],
        out_specs=pl.BlockSpec((tn, _D), lambda n: (n, 0)),
        out_shape=jax.ShapeDtypeStruct((_N, _D), jnp.float32),
    )(p0, p1, x, W_root, Wg, bias, bg)


def kernel(nodes, edges, edge_types, W_rel, W_root, bias, Wg, bg):
    x = nodes
    src = edges[0]
    dst = edges[1]
    et = edge_types

    xr = _compute_xr(x, W_rel).reshape(_R * _N, _D)
    zc = jnp.zeros((_SEG_P // _NS,), jnp.float32)
    cnt2 = _count(dst, et, zc).reshape(_NC, _SEG_P)
    inv = _compute_inv(cnt2).reshape(_SEG_P)
    zn = jnp.zeros((64, _D), jnp.float32)
    parts = _aggregate(xr, src, dst, et, inv, zn).reshape(_NC, _N, _D)
    return _final(parts[0], parts[1], x, W_root, Wg,
                  bias.reshape(1, _D), bg.reshape(1, _D))


# speedup vs baseline: 1.1612x; 1.0051x over previous
import functools

import jax
import jax.numpy as jnp
from jax import lax
from jax.experimental import pallas as pl
from jax.experimental.pallas import tpu as pltpu
from jax.experimental.pallas import tpu_sc as plsc

_N, _E, _D, _R = 10000, 320000, 128, 8
_NC, _NS = 2, 16
_NW = _NC * _NS
_EPW = _E // _NW
_CH = 80
_NCHUNK = _EPW // _CH
_SEGS = _N * _R
_SEG_P = 80128

_mesh = plsc.VectorSubcoreMesh(core_axis_name="c", subcore_axis_name="s")


def _xr_body(x_ref, w_ref, o_ref):
    o_ref[0] = jnp.dot(x_ref[...], w_ref[0],
                       preferred_element_type=jnp.float32)


def _compute_xr(x, W_rel):
    tn = 2000
    return pl.pallas_call(
        _xr_body,
        grid=(_R, _N // tn),
        in_specs=[
            pl.BlockSpec((tn, _D), lambda r, n: (n, 0)),
            pl.BlockSpec((1, _D, _D), lambda r, n: (r, 0, 0)),
        ],
        out_specs=pl.BlockSpec((1, tn, _D), lambda r, n: (r, n, 0)),
        out_shape=jax.ShapeDtypeStruct((_R, _N, _D), jnp.float32),
    )(x, W_rel)


def _count_body(dst_hbm, et_hbm, zc_hbm, cnt_out,
                dst_v, et_v, seg2d, ones_v, stage_v, cnt_sp):
    c = lax.axis_index("c")
    s = lax.axis_index("s")
    wid = c * _NS + s
    zsl = _SEG_P // _NS
    pltpu.sync_copy(zc_hbm.at[pl.ds(0, zsl)], stage_v)
    pltpu.sync_copy(stage_v, cnt_sp.at[pl.ds(s * zsl, zsl)])

    base = wid * _EPW
    pltpu.sync_copy(dst_hbm.at[pl.ds(base, _EPW)], dst_v)
    pltpu.sync_copy(et_hbm.at[pl.ds(base, _EPW)], et_v)

    def segstep(k, carry):
        sl = pl.ds(k * 16, 16)
        seg2d[0, sl] = dst_v[sl] * _R + et_v[sl]
        ones_v[sl] = jnp.ones((16,), jnp.float32)
        return carry

    lax.fori_loop(0, _EPW // 16, segstep, 0)
    plsc.subcore_barrier()
    pltpu.sync_copy(ones_v, cnt_sp.at[seg2d.at[0]], add=True)
    plsc.subcore_barrier()
    pltpu.sync_copy(cnt_sp.at[pl.ds(s * zsl, zsl)], stage_v)
    pltpu.sync_copy(stage_v, cnt_out.at[pl.ds(c * _SEG_P + s * zsl, zsl)])


_count = functools.partial(
    pl.kernel,
    out_type=jax.ShapeDtypeStruct((_NC * _SEG_P,), jnp.float32),
    mesh=_mesh,
    scratch_types=[
        pltpu.VMEM((_EPW,), jnp.int32),
        pltpu.VMEM((_EPW,), jnp.int32),
        pltpu.VMEM((1, _EPW), jnp.int32),
        pltpu.VMEM((_EPW,), jnp.float32),
        pltpu.VMEM((_SEG_P // _NS,), jnp.float32),
        pltpu.VMEM_SHARED((_SEG_P,), jnp.float32),
    ],
)(_count_body)


def _inv_body(c_ref, o_ref):
    c = c_ref[0] + c_ref[1]
    o_ref[...] = 1.0 / jnp.maximum(c, 1.0)


def _compute_inv(cnt2):
    return pl.pallas_call(
        _inv_body,
        out_shape=jax.ShapeDtypeStruct((_SEG_P // 128, 128), jnp.float32),
    )(cnt2.reshape(_NC, _SEG_P // 128, 128))


def _agg_body(xr_hbm, src_hbm, dst_hbm, et_hbm, inv_hbm, zn_hbm, out_hbm,
              rows0, rows1, srcm, etm, dstm, gidx3d, seg3d, dstA, dstB,
              w0, w1, stage_v, stage1d, semg0, semg1, semw0, semw1, semm,
              sems0, sems1, agg_sp, inv_sp):
    c = lax.axis_index("c")
    s = lax.axis_index("s")
    wid = c * _NS + s
    pltpu.sync_copy(zn_hbm, stage_v)
    for i in range(10):
        row0 = s * 640 + i * 64

        @pl.when(row0 + 64 <= _N)
        def _():
            pltpu.sync_copy(stage_v, agg_sp.at[pl.ds(row0, 64)])

    @pl.when(s == _NS - 1)
    def _():
        pltpu.sync_copy(stage_v.at[pl.ds(0, 16)],
                        agg_sp.at[pl.ds(_N - 16, 16)])

    zsl = _SEG_P // _NS
    pltpu.sync_copy(inv_hbm.at[pl.ds(s * zsl, zsl)], stage1d)
    pltpu.sync_copy(stage1d, inv_sp.at[pl.ds(s * zsl, zsl)])
    plsc.subcore_barrier()

    base = wid * _EPW
    rows = (rows0, rows1)
    semg = (semg0, semg1)
    semw = (semw0, semw1)
    w = (w0, w1)

    def fire_meta(j, p):
        off = base + j * 2 * _CH
        ring = pl.ds(p * 2 * _CH, 2 * _CH)
        pltpu.async_copy(src_hbm.at[pl.ds(off, 2 * _CH)], srcm.at[ring], semm)
        pltpu.async_copy(et_hbm.at[pl.ds(off, 2 * _CH)], etm.at[ring], semm)
        pltpu.async_copy(dst_hbm.at[pl.ds(off, 2 * _CH)], dstm.at[ring], semm)

    def wait_meta_index(j, p):
        off = base + j * 2 * _CH
        ring = pl.ds(p * 2 * _CH, 2 * _CH)
        pltpu.make_async_copy(src_hbm.at[pl.ds(off, 2 * _CH)], srcm.at[ring],
                              semm).wait()
        pltpu.make_async_copy(et_hbm.at[pl.ds(off, 2 * _CH)], etm.at[ring],
                              semm).wait()
        pltpu.make_async_copy(dst_hbm.at[pl.ds(off, 2 * _CH)], dstm.at[ring],
                              semm).wait()
        for q in range(2):
            for k in range(_CH // 16):
                sl = pl.ds(p * 2 * _CH + q * _CH + k * 16, 16)
                dsl = pl.ds(k * 16, 16)
                e = etm[sl]
                d = dstm[sl]
                gidx3d[p, q, dsl] = e * _N + srcm[sl]
                seg3d[p, q, dsl] = d * _R + e
                if k < 3:
                    dstA[p, q, pl.ds(k * 16, 16)] = d
                else:
                    dstB[p, q, pl.ds((k - 3) * 16, 16)] = d

    def fire_rows(b, p):
        pltpu.async_copy(xr_hbm.at[gidx3d.at[p, b]], rows[b], semg[b])
        pltpu.async_copy(inv_sp.at[seg3d.at[p, b]], w[b], semw[b])

    sems = (sems0, sems1)

    def _mult(b, k):
        w16 = w[b][pl.ds(k * 16, 16)]
        for jj in range(16):
            j = k * 16 + jj
            wv = w16[jj]
            for v in range(_D // 16):
                vsl = pl.ds(v * 16, 16)
                rows[b][j, vsl] = rows[b][j, vsl] * wv

    def consume(b, p):
        pltpu.make_async_copy(xr_hbm.at[gidx3d.at[p, b]], rows[b],
                              semg[b]).wait()
        pltpu.make_async_copy(inv_sp.at[seg3d.at[p, b]], w[b],
                              semw[b]).wait()
        for k in range(3):
            _mult(b, k)
        pltpu.async_copy(rows[b].at[pl.ds(0, 48)], agg_sp.at[dstA.at[p, b]],
                         sems[b], add=True)
        for k in range(3, _CH // 16):
            _mult(b, k)
        pltpu.async_copy(rows[b].at[pl.ds(48, 32)], agg_sp.at[dstB.at[p, b]],
                         sems[b], add=True)

    def wait_scatter(b, p):
        pltpu.make_async_copy(rows[b].at[pl.ds(0, 48)],
                              agg_sp.at[dstA.at[p, b]], sems[b]).wait()
        pltpu.make_async_copy(rows[b].at[pl.ds(48, 32)],
                              agg_sp.at[dstB.at[p, b]], sems[b]).wait()

    fire_meta(0, 0)
    wait_meta_index(0, 0)
    fire_rows(0, 0)
    fire_rows(1, 0)
    fire_meta(1, 1)

    def subbody(j, p):
        g = 2 * j
        consume(0, p)

        @pl.when(g + 2 < _NCHUNK)
        def _():
            wait_meta_index(j + 1, 1 - p)
            wait_scatter(0, p)
            fire_rows(0, 1 - p)

        @pl.when(g + 1 < _NCHUNK)
        def _():
            consume(1, p)

        @pl.when(2 * (j + 2) < _NCHUNK)
        def _():
            fire_meta(j + 2, p)

        @pl.when(g + 3 < _NCHUNK)
        def _():
            wait_scatter(1, p)
            fire_rows(1, 1 - p)

    def dpair(t, carry):
        subbody(2 * t, 0)
        subbody(2 * t + 1, 1)
        return carry

    lax.fori_loop(0, 31, dpair, 0)
    consume(0, 0)
    wait_scatter(1, 1)
    wait_scatter(0, 0)
    plsc.subcore_barrier()
    for i in range(10):
        row0 = s * 640 + i * 64

        @pl.when(row0 + 64 <= _N)
        def _():
            pltpu.sync_copy(agg_sp.at[pl.ds(row0, 64)], stage_v)
            pltpu.sync_copy(stage_v, out_hbm.at[pl.ds(c * _N + row0, 64)])

    @pl.when(s == _NS - 1)
    def _():
        pltpu.sync_copy(agg_sp.at[pl.ds(_N - 16, 16)], stage_v.at[pl.ds(0, 16)])
        pltpu.sync_copy(stage_v.at[pl.ds(0, 16)],
                        out_hbm.at[pl.ds(c * _N + _N - 16, 16)])


_aggregate = functools.partial(
    pl.kernel,
    out_type=jax.ShapeDtypeStruct((_NC * _N, _D), jnp.float32),
    mesh=_mesh,
    scratch_types=[
        pltpu.VMEM((_CH, _D), jnp.float32),
        pltpu.VMEM((_CH, _D), jnp.float32),
        pltpu.VMEM((4 * _CH,), jnp.int32),
        pltpu.VMEM((4 * _CH,), jnp.int32),
        pltpu.VMEM((4 * _CH,), jnp.int32),
        pltpu.VMEM((2, 2, _CH), jnp.int32),
        pltpu.VMEM((2, 2, _CH), jnp.int32),
        pltpu.VMEM((2, 2, 48), jnp.int32),
        pltpu.VMEM((2, 2, 32), jnp.int32),
        pltpu.VMEM((_CH,), jnp.float32),
        pltpu.VMEM((_CH,), jnp.float32),
        pltpu.VMEM((64, _D), jnp.float32),
        pltpu.VMEM((_SEG_P // _NS,), jnp.float32),
        pltpu.SemaphoreType.DMA,
        pltpu.SemaphoreType.DMA,
        pltpu.SemaphoreType.DMA,
        pltpu.SemaphoreType.DMA,
        pltpu.SemaphoreType.DMA,
        pltpu.SemaphoreType.DMA,
        pltpu.SemaphoreType.DMA,
        pltpu.VMEM_SHARED((_N, _D), jnp.float32),
        pltpu.VMEM_SHARED((_SEG_P,), jnp.float32),
    ],
    compiler_params=pltpu.CompilerParams(needs_layout_passes=False),
)(_agg_body)


def _final_body(p0, p1, x_ref, wr_ref, wg_ref, b_ref, bg_ref, o_ref):
    x = x_ref[...]
    nodes_ = (p0[...] + p1[...]
              + jnp.dot(x, wr_ref[...], preferred_element_type=jnp.float32)
              + b_ref[0])
    z = (jnp.dot(nodes_, wg_ref[0:_D, :], preferred_element_type=jnp.float32)
         + jnp.dot(x, wg_ref[_D:2 * _D, :], preferred_element_type=jnp.float32)
         + bg_ref[0])
    g = jax.nn.sigmoid(z)
    o_ref[...] = g * jnp.tanh(nodes_) + (1.0 - g) * x


def _final(p0, p1, x, W_root, Wg, bias, bg):
    tn = 2000
    full = lambda n: (0, 0)
    return pl.pallas_call(
        _final_body,
        grid=(_N // tn,),
        in_specs=[
            pl.BlockSpec((tn, _D), lambda n: (n, 0)),
            pl.BlockSpec((tn, _D), lambda n: (n, 0)),
            pl.BlockSpec((tn, _D), lambda n: (n, 0)),
            pl.BlockSpec((_D, _D), full),
            pl.BlockSpec((2 * _D, _D), full),
            pl.BlockSpec((1, _D), full),
            pl.BlockSpec((1, _D), full),
        ],
        out_specs=pl.BlockSpec((tn, _D), lambda n: (n, 0)),
        out_shape=jax.ShapeDtypeStruct((_N, _D), jnp.float32),
    )(p0, p1, x, W_root, Wg, bias, bg)


def kernel(nodes, edges, edge_types, W_rel, W_root, bias, Wg, bg):
    x = nodes
    src = edges[0]
    dst = edges[1]
    et = edge_types

    xr = _compute_xr(x, W_rel).reshape(_R * _N, _D)
    zc = jnp.zeros((_SEG_P // _NS,), jnp.float32)
    cnt2 = _count(dst, et, zc).reshape(_NC, _SEG_P)
    inv = _compute_inv(cnt2).reshape(_SEG_P)
    zn = jnp.zeros((64, _D), jnp.float32)
    parts = _aggregate(xr, src, dst, et, inv, zn).reshape(_NC, _N, _D)
    return _final(parts[0], parts[1], x, W_root, Wg,
                  bias.reshape(1, _D), bg.reshape(1, _D))
